# baseline (speedup 1.0000x reference)
import functools

import numpy as np
import jax
import jax.numpy as jnp
from jax import lax
from jax.experimental import pallas as pl
from jax.experimental.pallas import tpu as pltpu

_LANES = 128


def _apply_stage(x, CH, k, s, off):
    if s >= 8:
        nb = CH // (2 * s)
        x4 = x.reshape(nb, 2, s, _LANES)
        u = x4[:, 0]
        v = x4[:, 1]
        mn = jnp.minimum(u, v)
        mx = jnp.maximum(u, v)
        if k >= CH:
            asc = (off & k) == 0
            nu = jnp.where(asc, mn, mx)
            nv = jnp.where(asc, mx, mn)
        else:
            blk = lax.broadcasted_iota(jnp.int32, (nb, 1, _LANES), 0)
            pat = ((blk * (2 * s)) & k) == 0
            nu = jnp.where(pat, mn, mx)
            nv = jnp.where(pat, mx, mn)
        y = jnp.concatenate([nu[:, None], nv[:, None]], axis=1)
        return y.reshape(CH, _LANES)
    rows = lax.broadcasted_iota(jnp.int32, (CH, _LANES), 0)
    bit_clear = (rows & s) == 0
    p = jnp.where(bit_clear, jnp.roll(x, -s, axis=0), jnp.roll(x, s, axis=0))
    if k >= CH:
        asc = (off & k) == 0
    else:
        asc = (rows & k) == 0
    take_min = bit_clear == asc
    return jnp.where(take_min, jnp.minimum(x, p), jnp.maximum(x, p))


def _far_stage(scr, N, CH, k, s):
    ratio = s // CH

    def body(t, carry):
        q = t // ratio
        r = t - q * ratio
        u_off = q * (2 * s) + r * CH
        v_off = u_off + s
        u = scr[pl.ds(u_off, CH), :]
        v = scr[pl.ds(v_off, CH), :]
        mn = jnp.minimum(u, v)
        mx = jnp.maximum(u, v)
        asc = (u_off & k) == 0
        scr[pl.ds(u_off, CH), :] = jnp.where(asc, mn, mx)
        scr[pl.ds(v_off, CH), :] = jnp.where(asc, mx, mn)
        return carry

    lax.fori_loop(0, N // (2 * CH), body, 0)


def _chunk_pass(scr, N, CH, stages, first_mul=None, epilogue=None):
    def body(t, carry):
        off = t * CH
        if first_mul is None:
            x = scr[pl.ds(off, CH), :]
        else:
            x = first_mul(off, CH)
        for (k, s) in stages:
            x = _apply_stage(x, CH, k, s, off)
        if epilogue is None:
            scr[pl.ds(off, CH), :] = x
        else:
            epilogue(off, x)
        return carry

    lax.fori_loop(0, N // CH, body, 0)


def _sort_cols(scr, N, CH, first_mul, epilogue):
    init = []
    k = 2
    while k <= min(CH, N):
        s = k // 2
        while s > 0:
            init.append((k, s))
            s //= 2
        k *= 2
    _chunk_pass(scr, N, CH, init, first_mul=first_mul,
                epilogue=epilogue if k > N else None)
    while k <= N:
        s = k // 2
        while s >= CH:
            _far_stage(scr, N, CH, k, s)
            s //= 2
        tail = []
        while s > 0:
            tail.append((k, s))
            s //= 2
        _chunk_pass(scr, N, CH, tail,
                    epilogue=epilogue if k == N else None)
        k *= 2


def _swd_kernel(N, CH, C, B, x_ref, m_ref, out_ref, scr_cur, scr_keep):
    g = pl.program_id(0)
    j = g % 2
    nch = N // CH

    c0 = (g // 2) * _LANES
    lane = lax.broadcasted_iota(jnp.int32, (B, _LANES), 1)
    bidx = (c0 + lane) // C
    brow = lax.broadcasted_iota(jnp.int32, (B, _LANES), 0)
    onehot = (bidx == brow).astype(jnp.float32)

    def first_mul(off, ch):
        msel = jnp.dot(m_ref[pl.ds(off, ch), :], onehot,
                       preferred_element_type=jnp.float32)
        return x_ref[0, pl.ds(off, ch), :] * msel

    _sort_cols(scr_cur, N, CH, first_mul, None)

    @pl.when(j == 0)
    def _():
        def cp_body(t, carry):
            scr_keep[pl.ds(t * CH, CH), :] = scr_cur[pl.ds(t * CH, CH), :]
            return carry

        lax.fori_loop(0, nch, cp_body, 0)

    @pl.when(j == 1)
    def _():
        def acc_body(t, acc):
            d = jnp.abs(scr_cur[pl.ds(t * CH, CH), :] -
                        scr_keep[pl.ds(t * CH, CH), :])
            return acc + jnp.sum(d, axis=0, keepdims=True)

        out_ref[0] = lax.fori_loop(0, nch, acc_body,
                                   jnp.zeros((1, _LANES), jnp.float32))


def _scale_colsums(f1, f2, um, CH=256):
    B, C, h, w = f1.shape
    stride = um.shape[1] // h
    N = h * w
    BC = B * C
    m = um[:, ::stride, ::stride].reshape(B, N)
    a_t = f1.reshape(BC, N).T
    b_t = f2.reshape(BC, N).T
    x = jnp.stack([a_t, b_t])
    ncb = BC // _LANES
    out = pl.pallas_call(
        functools.partial(_swd_kernel, N, CH, C, B),
        grid=(2 * ncb,),
        in_specs=[
            pl.BlockSpec((1, N, _LANES), lambda g: (g % 2, 0, g // 2)),
            pl.BlockSpec((N, B), lambda g: (0, 0)),
        ],
        out_specs=pl.BlockSpec((1, 1, _LANES), lambda g: (g // 2, 0, 0)),
        out_shape=jax.ShapeDtypeStruct((ncb, 1, _LANES), jnp.float32),
        scratch_shapes=[
            pltpu.VMEM((N, _LANES), jnp.float32),
            pltpu.VMEM((N, _LANES), jnp.float32),
        ],
    )(x, m.T)
    return out.reshape(BC), m, N


@jax.jit
def kernel(feat_t1_s0, feat_t1_s1, feat_t2_s0, feat_t2_s1, target_mask):
    um = (1 - target_mask).astype(jnp.float32)
    losses = []
    for f1, f2 in ((feat_t1_s0, feat_t2_s0), (feat_t1_s1, feat_t2_s1)):
        colsums, m, N = _scale_colsums(f1, f2, um)
        B, C = f1.shape[0], f1.shape[1]
        valid = jnp.maximum(jnp.sum(m, axis=1), 1.0)
        per_b = colsums.reshape(B, C).sum(axis=1) / valid
        losses.append(jnp.sum(per_b) / (B * C * N))
    return (losses[0] + losses[1]) * 0.5


# speedup vs baseline: 1.0530x; 1.0530x over previous
import functools

import numpy as np
import jax
import jax.numpy as jnp
from jax import lax
from jax.experimental import pallas as pl
from jax.experimental.pallas import tpu as pltpu

_LANES = 128


def _apply_stage(x, CH, k, s, off):
    if s >= 8:
        nb = CH // (2 * s)
        x4 = x.reshape(nb, 2, s, _LANES)
        u = x4[:, 0]
        v = x4[:, 1]
        mn = jnp.minimum(u, v)
        mx = jnp.maximum(u, v)
        if k >= CH:
            asc = (off & k) == 0
            nu = jnp.where(asc, mn, mx)
            nv = jnp.where(asc, mx, mn)
        else:
            blk = lax.broadcasted_iota(jnp.int32, (nb, 1, _LANES), 0)
            pat = ((blk * (2 * s)) & k) == 0
            nu = jnp.where(pat, mn, mx)
            nv = jnp.where(pat, mx, mn)
        y = jnp.concatenate([nu[:, None], nv[:, None]], axis=1)
        return y.reshape(CH, _LANES)
    rows = lax.broadcasted_iota(jnp.int32, (CH, _LANES), 0)
    bit_clear = (rows & s) == 0
    p = jnp.where(bit_clear, jnp.roll(x, -s, axis=0), jnp.roll(x, s, axis=0))
    if k >= CH:
        asc = (off & k) == 0
    else:
        asc = (rows & k) == 0
    take_min = bit_clear == asc
    return jnp.where(take_min, jnp.minimum(x, p), jnp.maximum(x, p))


def _far_stage(scr, N, CH, k, s):
    ratio = s // CH

    def body(t, carry):
        q = t // ratio
        r = t - q * ratio
        u_off = q * (2 * s) + r * CH
        v_off = u_off + s
        u = scr[pl.ds(u_off, CH), :]
        v = scr[pl.ds(v_off, CH), :]
        mn = jnp.minimum(u, v)
        mx = jnp.maximum(u, v)
        asc = (u_off & k) == 0
        scr[pl.ds(u_off, CH), :] = jnp.where(asc, mn, mx)
        scr[pl.ds(v_off, CH), :] = jnp.where(asc, mx, mn)
        return carry

    lax.fori_loop(0, N // (2 * CH), body, 0)


def _chunk_pass(scr, N, CH, stages, first_mul=None, epilogue=None):
    def body(t, carry):
        off = t * CH
        if first_mul is None:
            x = scr[pl.ds(off, CH), :]
        else:
            x = first_mul(off, CH)
        for (k, s) in stages:
            x = _apply_stage(x, CH, k, s, off)
        if epilogue is None:
            scr[pl.ds(off, CH), :] = x
        else:
            epilogue(off, x)
        return carry

    lax.fori_loop(0, N // CH, body, 0)


def _sort_cols(scr, N, CH, first_mul, epilogue):
    init = []
    k = 2
    while k <= min(CH, N):
        s = k // 2
        while s > 0:
            init.append((k, s))
            s //= 2
        k *= 2
    _chunk_pass(scr, N, CH, init, first_mul=first_mul,
                epilogue=epilogue if k > N else None)
    while k <= N:
        s = k // 2
        while s >= CH:
            _far_stage(scr, N, CH, k, s)
            s //= 2
        tail = []
        while s > 0:
            tail.append((k, s))
            s //= 2
        _chunk_pass(scr, N, CH, tail,
                    epilogue=epilogue if k == N else None)
        k *= 2


def _swd_kernel(N, CH, x_ref, m_ref, out_ref, scr_cur, scr_keep):
    j = pl.program_id(0) % 2
    nch = N // CH

    def first_mul(off, ch):
        return x_ref[0, pl.ds(off, ch), :] * m_ref[pl.ds(off, ch), :]

    _sort_cols(scr_cur, N, CH, first_mul, None)

    @pl.when(j == 0)
    def _():
        def cp_body(t, carry):
            scr_keep[pl.ds(t * CH, CH), :] = scr_cur[pl.ds(t * CH, CH), :]
            return carry

        lax.fori_loop(0, nch, cp_body, 0)

    @pl.when(j == 1)
    def _():
        def acc_body(t, acc):
            d = jnp.abs(scr_cur[pl.ds(t * CH, CH), :] -
                        scr_keep[pl.ds(t * CH, CH), :])
            return acc + jnp.sum(d, axis=0, keepdims=True)

        out_ref[0] = lax.fori_loop(0, nch, acc_body,
                                   jnp.zeros((1, _LANES), jnp.float32))


def _scale_colsums(f1, f2, um, CH=256):
    B, C, h, w = f1.shape
    stride = um.shape[1] // h
    N = h * w
    BC = B * C
    m = um[:, ::stride, ::stride].reshape(B, N)
    m_bc = jnp.repeat(m.T, C, axis=1)
    a_t = f1.reshape(BC, N).T
    b_t = f2.reshape(BC, N).T
    x = jnp.stack([a_t, b_t])
    ncb = BC // _LANES
    out = pl.pallas_call(
        functools.partial(_swd_kernel, N, CH),
        grid=(2 * ncb,),
        in_specs=[
            pl.BlockSpec((1, N, _LANES), lambda g: (g % 2, 0, g // 2)),
            pl.BlockSpec((N, _LANES), lambda g: (0, g // 2)),
        ],
        out_specs=pl.BlockSpec((1, 1, _LANES), lambda g: (g // 2, 0, 0)),
        out_shape=jax.ShapeDtypeStruct((ncb, 1, _LANES), jnp.float32),
        scratch_shapes=[
            pltpu.VMEM((N, _LANES), jnp.float32),
            pltpu.VMEM((N, _LANES), jnp.float32),
        ],
    )(x, m_bc)
    return out.reshape(BC), m, N


@jax.jit
def kernel(feat_t1_s0, feat_t1_s1, feat_t2_s0, feat_t2_s1, target_mask):
    um = (1 - target_mask).astype(jnp.float32)
    losses = []
    for f1, f2 in ((feat_t1_s0, feat_t2_s0), (feat_t1_s1, feat_t2_s1)):
        colsums, m, N = _scale_colsums(f1, f2, um)
        B, C = f1.shape[0], f1.shape[1]
        valid = jnp.maximum(jnp.sum(m, axis=1), 1.0)
        per_b = colsums.reshape(B, C).sum(axis=1) / valid
        losses.append(jnp.sum(per_b) / (B * C * N))
    return (losses[0] + losses[1]) * 0.5


# speedup vs baseline: 1.0572x; 1.0040x over previous
import functools

import numpy as np
import jax
import jax.numpy as jnp
from jax import lax
from jax.experimental import pallas as pl
from jax.experimental.pallas import tpu as pltpu

_LANES = 128


def _stage_masks(CH, stages):
    masks = []
    for (k, s) in stages:
        if s >= 8:
            if k >= CH:
                masks.append(None)
            else:
                nb = CH // (2 * s)
                blk = lax.broadcasted_iota(jnp.int32, (nb, 1, _LANES), 0)
                masks.append(((blk * (2 * s)) & k) == 0)
        else:
            rows = lax.broadcasted_iota(jnp.int32, (CH, _LANES), 0)
            bit_clear = (rows & s) == 0
            ascv = None if k >= CH else ((rows & k) == 0)
            masks.append((bit_clear, ascv))
    return masks


def _apply_stage(x, CH, k, s, off, mask):
    if s >= 8:
        nb = CH // (2 * s)
        x4 = x.reshape(nb, 2, s, _LANES)
        u = x4[:, 0]
        v = x4[:, 1]
        mn = jnp.minimum(u, v)
        mx = jnp.maximum(u, v)
        if k >= CH:
            asc = (off & k) == 0
            nu = jnp.where(asc, mn, mx)
            nv = jnp.where(asc, mx, mn)
        else:
            nu = jnp.where(mask, mn, mx)
            nv = jnp.where(mask, mx, mn)
        y = jnp.concatenate([nu[:, None], nv[:, None]], axis=1)
        return y.reshape(CH, _LANES)
    bit_clear, ascv = mask
    p = jnp.where(bit_clear, jnp.roll(x, -s, axis=0), jnp.roll(x, s, axis=0))
    if k >= CH:
        asc = (off & k) == 0
    else:
        asc = ascv
    take_min = bit_clear == asc
    return jnp.where(take_min, jnp.minimum(x, p), jnp.maximum(x, p))


def _far_stage(scr, N, CH, k, s):
    ratio = s // CH

    def body(t, carry):
        q = t // ratio
        r = t - q * ratio
        u_off = q * (2 * s) + r * CH
        v_off = u_off + s
        u = scr[pl.ds(u_off, CH), :]
        v = scr[pl.ds(v_off, CH), :]
        mn = jnp.minimum(u, v)
        mx = jnp.maximum(u, v)
        asc = (u_off & k) == 0
        scr[pl.ds(u_off, CH), :] = jnp.where(asc, mn, mx)
        scr[pl.ds(v_off, CH), :] = jnp.where(asc, mx, mn)
        return carry

    lax.fori_loop(0, N // (2 * CH), body, 0)


def _chunk_pass(scr, N, CH, stages, first_mul=None, epilogue=None):
    masks = _stage_masks(CH, stages)

    def body(t, carry):
        off = t * CH
        if first_mul is None:
            x = scr[pl.ds(off, CH), :]
        else:
            x = first_mul(off, CH)
        for (k, s), mask in zip(stages, masks):
            x = _apply_stage(x, CH, k, s, off, mask)
        if epilogue is None:
            scr[pl.ds(off, CH), :] = x
        else:
            epilogue(off, x)
        return carry

    lax.fori_loop(0, N // CH, body, 0)


def _sort_cols(scr, N, CH, first_mul, epilogue):
    init = []
    k = 2
    while k <= min(CH, N):
        s = k // 2
        while s > 0:
            init.append((k, s))
            s //= 2
        k *= 2
    _chunk_pass(scr, N, CH, init, first_mul=first_mul,
                epilogue=epilogue if k > N else None)
    while k <= N:
        s = k // 2
        while s >= CH:
            _far_stage(scr, N, CH, k, s)
            s //= 2
        tail = []
        while s > 0:
            tail.append((k, s))
            s //= 2
        _chunk_pass(scr, N, CH, tail,
                    epilogue=epilogue if k == N else None)
        k *= 2


def _swd_kernel(N, CH, x_ref, m_ref, out_ref, scr_cur, scr_keep):
    j = pl.program_id(0) % 2
    nch = N // CH

    def first_mul(off, ch):
        return x_ref[0, pl.ds(off, ch), :] * m_ref[pl.ds(off, ch), :]

    _sort_cols(scr_cur, N, CH, first_mul, None)

    @pl.when(j == 0)
    def _():
        def cp_body(t, carry):
            scr_keep[pl.ds(t * CH, CH), :] = scr_cur[pl.ds(t * CH, CH), :]
            return carry

        lax.fori_loop(0, nch, cp_body, 0)

    @pl.when(j == 1)
    def _():
        def acc_body(t, acc):
            d = jnp.abs(scr_cur[pl.ds(t * CH, CH), :] -
                        scr_keep[pl.ds(t * CH, CH), :])
            return acc + jnp.sum(d, axis=0, keepdims=True)

        out_ref[0] = lax.fori_loop(0, nch, acc_body,
                                   jnp.zeros((1, _LANES), jnp.float32))


def _scale_colsums(f1, f2, um, CH=256):
    B, C, h, w = f1.shape
    stride = um.shape[1] // h
    N = h * w
    BC = B * C
    m = um[:, ::stride, ::stride].reshape(B, N)
    m_bc = jnp.repeat(m.T, C, axis=1)
    a_t = f1.reshape(BC, N).T
    b_t = f2.reshape(BC, N).T
    x = jnp.stack([a_t, b_t])
    ncb = BC // _LANES
    out = pl.pallas_call(
        functools.partial(_swd_kernel, N, CH),
        grid=(2 * ncb,),
        in_specs=[
            pl.BlockSpec((1, N, _LANES), lambda g: (g % 2, 0, g // 2)),
            pl.BlockSpec((N, _LANES), lambda g: (0, g // 2)),
        ],
        out_specs=pl.BlockSpec((1, 1, _LANES), lambda g: (g // 2, 0, 0)),
        out_shape=jax.ShapeDtypeStruct((ncb, 1, _LANES), jnp.float32),
        scratch_shapes=[
            pltpu.VMEM((N, _LANES), jnp.float32),
            pltpu.VMEM((N, _LANES), jnp.float32),
        ],
    )(x, m_bc)
    return out.reshape(BC), m, N


@jax.jit
def kernel(feat_t1_s0, feat_t1_s1, feat_t2_s0, feat_t2_s1, target_mask):
    um = (1 - target_mask).astype(jnp.float32)
    losses = []
    for f1, f2 in ((feat_t1_s0, feat_t2_s0), (feat_t1_s1, feat_t2_s1)):
        colsums, m, N = _scale_colsums(f1, f2, um)
        B, C = f1.shape[0], f1.shape[1]
        valid = jnp.maximum(jnp.sum(m, axis=1), 1.0)
        per_b = colsums.reshape(B, C).sum(axis=1) / valid
        losses.append(jnp.sum(per_b) / (B * C * N))
    return (losses[0] + losses[1]) * 0.5


# speedup vs baseline: 1.2395x; 1.1725x over previous
import functools

import numpy as np
import jax
import jax.numpy as jnp
from jax import lax
from jax.experimental import pallas as pl
from jax.experimental.pallas import tpu as pltpu

_LANES = 128


def _stage_masks(CH, stages):
    masks = []
    for (k, s) in stages:
        if s >= 8:
            if k >= CH:
                masks.append(None)
            else:
                nb = CH // (2 * s)
                blk = lax.broadcasted_iota(jnp.int32, (nb, 1, _LANES), 0)
                masks.append(((blk * (2 * s)) & k) == 0)
        else:
            rows = lax.broadcasted_iota(jnp.int32, (CH, _LANES), 0)
            bit_clear = (rows & s) == 0
            if k >= CH:
                pat = None
            else:
                pat = bit_clear == ((rows & k) == 0)
            masks.append((bit_clear, jnp.logical_not(bit_clear), pat))
    return masks


def _apply_stage(x, CH, s, mode, mask):
    if s >= 8:
        nb = CH // (2 * s)
        x4 = x.reshape(nb, 2, s, _LANES)
        u = x4[:, 0]
        v = x4[:, 1]
        mn = jnp.minimum(u, v)
        mx = jnp.maximum(u, v)
        if mode == "pat":
            nu = jnp.where(mask, mn, mx)
            nv = jnp.where(mask, mx, mn)
        elif mode == "asc":
            nu, nv = mn, mx
        else:
            nu, nv = mx, mn
        y = jnp.concatenate([nu[:, None], nv[:, None]], axis=1)
        return y.reshape(CH, _LANES)
    bit_clear, not_bit_clear, pat = mask
    p = jnp.where(bit_clear, jnp.roll(x, -s, axis=0), jnp.roll(x, s, axis=0))
    if mode == "pat":
        take_min = pat
    elif mode == "asc":
        take_min = bit_clear
    else:
        take_min = not_bit_clear
    return jnp.where(take_min, jnp.minimum(x, p), jnp.maximum(x, p))


def _far_stage(scr, N, CH, k, s):
    ratio = s // CH
    w = k // (2 * s)

    def run(asc, full):
        def body(t, carry):
            qp = t // ratio
            r = t - qp * ratio
            if full:
                q = qp
            else:
                qh = qp // w
                q = qh * (2 * w) + (qp - qh * w)
                if not asc:
                    q = q + w
            u_off = q * (2 * s) + r * CH
            v_off = u_off + s
            u = scr[pl.ds(u_off, CH), :]
            v = scr[pl.ds(v_off, CH), :]
            mn = jnp.minimum(u, v)
            mx = jnp.maximum(u, v)
            if asc:
                scr[pl.ds(u_off, CH), :] = mn
                scr[pl.ds(v_off, CH), :] = mx
            else:
                scr[pl.ds(u_off, CH), :] = mx
                scr[pl.ds(v_off, CH), :] = mn
            return carry

        trips = N // (2 * CH) if full else N // (4 * CH)
        lax.fori_loop(0, trips, body, 0)

    if k == N:
        run(True, True)
    else:
        run(True, False)
        run(False, False)


def _chunk_pass(scr, N, CH, stages, k_dir, first_mul=None, epilogue=None):
    masks = _stage_masks(CH, stages)
    w = k_dir // CH

    def run(dir_mode, trips, tmap):
        def body(t_p, carry):
            t = tmap(t_p)
            off = t * CH
            if first_mul is None:
                x = scr[pl.ds(off, CH), :]
            else:
                x = first_mul(off, CH)
            for (k, s), mask in zip(stages, masks):
                mode = dir_mode if k >= CH else "pat"
                x = _apply_stage(x, CH, s, mode, mask)
            if epilogue is None:
                scr[pl.ds(off, CH), :] = x
            else:
                epilogue(off, x)
            return carry

        lax.fori_loop(0, trips, body, 0)

    if k_dir == N:
        run("asc", N // CH, lambda t_p: t_p)
    else:
        run("asc", N // (2 * CH),
            lambda t_p: (t_p // w) * (2 * w) + (t_p - (t_p // w) * w))
        run("desc", N // (2 * CH),
            lambda t_p: (t_p // w) * (2 * w) + (t_p - (t_p // w) * w) + w)


def _sort_cols(scr, N, CH, first_mul, epilogue):
    init = []
    k = 2
    while k <= min(CH, N):
        s = k // 2
        while s > 0:
            init.append((k, s))
            s //= 2
        k *= 2
    _chunk_pass(scr, N, CH, init, min(CH, N), first_mul=first_mul,
                epilogue=epilogue if k > N else None)
    while k <= N:
        s = k // 2
        while s >= CH:
            _far_stage(scr, N, CH, k, s)
            s //= 2
        tail = []
        while s > 0:
            tail.append((k, s))
            s //= 2
        _chunk_pass(scr, N, CH, tail, k,
                    epilogue=epilogue if k == N else None)
        k *= 2


def _swd_kernel(N, CH, x_ref, m_ref, out_ref, scr_cur, scr_keep):
    j = pl.program_id(0) % 2
    nch = N // CH

    def first_mul(off, ch):
        return x_ref[0, pl.ds(off, ch), :] * m_ref[pl.ds(off, ch), :]

    _sort_cols(scr_cur, N, CH, first_mul, None)

    @pl.when(j == 0)
    def _():
        def cp_body(t, carry):
            scr_keep[pl.ds(t * CH, CH), :] = scr_cur[pl.ds(t * CH, CH), :]
            return carry

        lax.fori_loop(0, nch, cp_body, 0)

    @pl.when(j == 1)
    def _():
        def acc_body(t, acc):
            d = jnp.abs(scr_cur[pl.ds(t * CH, CH), :] -
                        scr_keep[pl.ds(t * CH, CH), :])
            return acc + jnp.sum(d, axis=0, keepdims=True)

        out_ref[0] = lax.fori_loop(0, nch, acc_body,
                                   jnp.zeros((1, _LANES), jnp.float32))


def _scale_colsums(f1, f2, um, CH=256):
    B, C, h, w = f1.shape
    stride = um.shape[1] // h
    N = h * w
    BC = B * C
    m = um[:, ::stride, ::stride].reshape(B, N)
    m_bc = jnp.repeat(m.T, C, axis=1)
    a_t = f1.reshape(BC, N).T
    b_t = f2.reshape(BC, N).T
    x = jnp.stack([a_t, b_t])
    ncb = BC // _LANES
    out = pl.pallas_call(
        functools.partial(_swd_kernel, N, CH),
        grid=(2 * ncb,),
        in_specs=[
            pl.BlockSpec((1, N, _LANES), lambda g: (g % 2, 0, g // 2)),
            pl.BlockSpec((N, _LANES), lambda g: (0, g // 2)),
        ],
        out_specs=pl.BlockSpec((1, 1, _LANES), lambda g: (g // 2, 0, 0)),
        out_shape=jax.ShapeDtypeStruct((ncb, 1, _LANES), jnp.float32),
        scratch_shapes=[
            pltpu.VMEM((N, _LANES), jnp.float32),
            pltpu.VMEM((N, _LANES), jnp.float32),
        ],
    )(x, m_bc)
    return out.reshape(BC), m, N


@jax.jit
def kernel(feat_t1_s0, feat_t1_s1, feat_t2_s0, feat_t2_s1, target_mask):
    um = (1 - target_mask).astype(jnp.float32)
    losses = []
    for f1, f2 in ((feat_t1_s0, feat_t2_s0), (feat_t1_s1, feat_t2_s1)):
        colsums, m, N = _scale_colsums(f1, f2, um)
        B, C = f1.shape[0], f1.shape[1]
        valid = jnp.maximum(jnp.sum(m, axis=1), 1.0)
        per_b = colsums.reshape(B, C).sum(axis=1) / valid
        losses.append(jnp.sum(per_b) / (B * C * N))
    return (losses[0] + losses[1]) * 0.5


# speedup vs baseline: 1.2711x; 1.0254x over previous
import functools

import numpy as np
import jax
import jax.numpy as jnp
from jax import lax
from jax.experimental import pallas as pl
from jax.experimental.pallas import tpu as pltpu

_LANES = 128


def _stage_masks(CH, stages):
    masks = []
    for (k, s) in stages:
        if s >= 8:
            if k >= CH:
                masks.append(None)
            else:
                nb = CH // (2 * s)
                blk = lax.broadcasted_iota(jnp.int32, (nb, 1, _LANES), 0)
                masks.append(((blk * (2 * s)) & k) == 0)
        else:
            rows = lax.broadcasted_iota(jnp.int32, (CH, _LANES), 0)
            bit_clear = (rows & s) == 0
            if k >= CH:
                pat = None
            else:
                pat = bit_clear == ((rows & k) == 0)
            masks.append((bit_clear, jnp.logical_not(bit_clear), pat))
    return masks


def _apply_stage(x, CH, s, mode, mask):
    if s >= 8:
        nb = CH // (2 * s)
        x4 = x.reshape(nb, 2, s, _LANES)
        u = x4[:, 0]
        v = x4[:, 1]
        mn = jnp.minimum(u, v)
        mx = jnp.maximum(u, v)
        if mode == "pat":
            nu = jnp.where(mask, mn, mx)
            nv = jnp.where(mask, mx, mn)
        elif mode == "asc":
            nu, nv = mn, mx
        else:
            nu, nv = mx, mn
        y = jnp.concatenate([nu[:, None], nv[:, None]], axis=1)
        return y.reshape(CH, _LANES)
    bit_clear, not_bit_clear, pat = mask
    p = jnp.where(bit_clear, jnp.roll(x, -s, axis=0), jnp.roll(x, s, axis=0))
    if mode == "pat":
        take_min = pat
    elif mode == "asc":
        take_min = bit_clear
    else:
        take_min = not_bit_clear
    return jnp.where(take_min, jnp.minimum(x, p), jnp.maximum(x, p))


def _far_stage(scr, N, CH, k, s):
    ratio = s // CH
    w = k // (2 * s)

    def run(asc, full):
        def body(t, carry):
            qp = t // ratio
            r = t - qp * ratio
            if full:
                q = qp
            else:
                qh = qp // w
                q = qh * (2 * w) + (qp - qh * w)
                if not asc:
                    q = q + w
            u_off = q * (2 * s) + r * CH
            v_off = u_off + s
            u = scr[pl.ds(u_off, CH), :]
            v = scr[pl.ds(v_off, CH), :]
            mn = jnp.minimum(u, v)
            mx = jnp.maximum(u, v)
            if asc:
                scr[pl.ds(u_off, CH), :] = mn
                scr[pl.ds(v_off, CH), :] = mx
            else:
                scr[pl.ds(u_off, CH), :] = mx
                scr[pl.ds(v_off, CH), :] = mn
            return carry

        trips = N // (2 * CH) if full else N // (4 * CH)
        lax.fori_loop(0, trips, body, 0)

    if k == N:
        run(True, True)
    else:
        run(True, False)
        run(False, False)


def _far_pair(scr, N, CHF, k, s1):
    s2 = s1 // 2
    nq = s1 // (2 * CHF)
    nc = s2 // CHF
    w = k // (2 * s1)

    def run(asc, full, trips):
        def body(t, carry):
            qp = t // nq
            rem = t - qp * nq
            rr = rem // nc
            cc = rem - rr * nc
            if full:
                q = qp
            else:
                qh = qp // w
                q = qh * (2 * w) + (qp - qh * w)
                if not asc:
                    q = q + w
            base = q * (2 * s1) + rr * (2 * s2) + cc * CHF
            a = scr[pl.ds(base, CHF), :]
            b = scr[pl.ds(base + s2, CHF), :]
            c = scr[pl.ds(base + s1, CHF), :]
            d = scr[pl.ds(base + s1 + s2, CHF), :]
            if asc:
                a, c = jnp.minimum(a, c), jnp.maximum(a, c)
                b, d = jnp.minimum(b, d), jnp.maximum(b, d)
                a, b = jnp.minimum(a, b), jnp.maximum(a, b)
                c, d = jnp.minimum(c, d), jnp.maximum(c, d)
            else:
                a, c = jnp.maximum(a, c), jnp.minimum(a, c)
                b, d = jnp.maximum(b, d), jnp.minimum(b, d)
                a, b = jnp.maximum(a, b), jnp.minimum(a, b)
                c, d = jnp.maximum(c, d), jnp.minimum(c, d)
            scr[pl.ds(base, CHF), :] = a
            scr[pl.ds(base + s2, CHF), :] = b
            scr[pl.ds(base + s1, CHF), :] = c
            scr[pl.ds(base + s1 + s2, CHF), :] = d
            return carry

        lax.fori_loop(0, trips, body, 0)

    if k == N:
        run(True, True, N // (4 * CHF))
    else:
        run(True, False, N // (8 * CHF))
        run(False, False, N // (8 * CHF))


def _chunk_pass(scr, N, CH, stages, k_dir, first_mul=None, epilogue=None):
    masks = _stage_masks(CH, stages)
    w = k_dir // CH

    def run(dir_mode, trips, tmap):
        def body(t_p, carry):
            t = tmap(t_p)
            off = t * CH
            if first_mul is None:
                x = scr[pl.ds(off, CH), :]
            else:
                x = first_mul(off, CH)
            for (k, s), mask in zip(stages, masks):
                mode = dir_mode if k >= CH else "pat"
                x = _apply_stage(x, CH, s, mode, mask)
            if epilogue is None:
                scr[pl.ds(off, CH), :] = x
            else:
                epilogue(off, x)
            return carry

        lax.fori_loop(0, trips, body, 0)

    if k_dir == N:
        run("asc", N // CH, lambda t_p: t_p)
    else:
        run("asc", N // (2 * CH),
            lambda t_p: (t_p // w) * (2 * w) + (t_p - (t_p // w) * w))
        run("desc", N // (2 * CH),
            lambda t_p: (t_p // w) * (2 * w) + (t_p - (t_p // w) * w) + w)


def _sort_cols(scr, N, CH, first_mul, epilogue):
    init = []
    k = 2
    while k <= min(CH, N):
        s = k // 2
        while s > 0:
            init.append((k, s))
            s //= 2
        k *= 2
    _chunk_pass(scr, N, CH, init, min(CH, N), first_mul=first_mul,
                epilogue=epilogue if k > N else None)
    while k <= N:
        far = []
        s = k // 2
        while s >= CH:
            far.append(s)
            s //= 2
        i = 0
        while i + 1 < len(far):
            _far_pair(scr, N, min(128, CH), k, far[i])
            i += 2
        if i < len(far):
            _far_stage(scr, N, CH, k, far[i])
        tail = []
        while s > 0:
            tail.append((k, s))
            s //= 2
        _chunk_pass(scr, N, CH, tail, k,
                    epilogue=epilogue if k == N else None)
        k *= 2


def _swd_kernel(N, CH, x_ref, m_ref, out_ref, scr_cur, scr_keep):
    j = pl.program_id(0) % 2
    nch = N // CH

    def first_mul(off, ch):
        return x_ref[0, pl.ds(off, ch), :] * m_ref[pl.ds(off, ch), :]

    _sort_cols(scr_cur, N, CH, first_mul, None)

    @pl.when(j == 0)
    def _():
        def cp_body(t, carry):
            scr_keep[pl.ds(t * CH, CH), :] = scr_cur[pl.ds(t * CH, CH), :]
            return carry

        lax.fori_loop(0, nch, cp_body, 0)

    @pl.when(j == 1)
    def _():
        def acc_body(t, acc):
            d = jnp.abs(scr_cur[pl.ds(t * CH, CH), :] -
                        scr_keep[pl.ds(t * CH, CH), :])
            return acc + jnp.sum(d, axis=0, keepdims=True)

        out_ref[0] = lax.fori_loop(0, nch, acc_body,
                                   jnp.zeros((1, _LANES), jnp.float32))


def _scale_colsums(f1, f2, um, CH=256):
    B, C, h, w = f1.shape
    stride = um.shape[1] // h
    N = h * w
    BC = B * C
    m = um[:, ::stride, ::stride].reshape(B, N)
    m_bc = jnp.repeat(m.T, C, axis=1)
    a_t = f1.reshape(BC, N).T
    b_t = f2.reshape(BC, N).T
    x = jnp.stack([a_t, b_t])
    ncb = BC // _LANES
    out = pl.pallas_call(
        functools.partial(_swd_kernel, N, CH),
        grid=(2 * ncb,),
        in_specs=[
            pl.BlockSpec((1, N, _LANES), lambda g: (g % 2, 0, g // 2)),
            pl.BlockSpec((N, _LANES), lambda g: (0, g // 2)),
        ],
        out_specs=pl.BlockSpec((1, 1, _LANES), lambda g: (g // 2, 0, 0)),
        out_shape=jax.ShapeDtypeStruct((ncb, 1, _LANES), jnp.float32),
        scratch_shapes=[
            pltpu.VMEM((N, _LANES), jnp.float32),
            pltpu.VMEM((N, _LANES), jnp.float32),
        ],
    )(x, m_bc)
    return out.reshape(BC), m, N


@jax.jit
def kernel(feat_t1_s0, feat_t1_s1, feat_t2_s0, feat_t2_s1, target_mask):
    um = (1 - target_mask).astype(jnp.float32)
    losses = []
    for f1, f2 in ((feat_t1_s0, feat_t2_s0), (feat_t1_s1, feat_t2_s1)):
        colsums, m, N = _scale_colsums(f1, f2, um)
        B, C = f1.shape[0], f1.shape[1]
        valid = jnp.maximum(jnp.sum(m, axis=1), 1.0)
        per_b = colsums.reshape(B, C).sum(axis=1) / valid
        losses.append(jnp.sum(per_b) / (B * C * N))
    return (losses[0] + losses[1]) * 0.5


# speedup vs baseline: 1.2718x; 1.0006x over previous
import functools

import numpy as np
import jax
import jax.numpy as jnp
from jax import lax
from jax.experimental import pallas as pl
from jax.experimental.pallas import tpu as pltpu

_LANES = 128


def _stage_masks(CH, stages):
    masks = []
    for (k, s) in stages:
        if s >= 8:
            if k >= CH:
                masks.append(None)
            else:
                nb = CH // (2 * s)
                blk = lax.broadcasted_iota(jnp.int32, (nb, 1, _LANES), 0)
                masks.append(((blk * (2 * s)) & k) == 0)
        else:
            rows = lax.broadcasted_iota(jnp.int32, (CH, _LANES), 0)
            bit_clear = (rows & s) == 0
            if k >= CH:
                pat = None
            else:
                pat = bit_clear == ((rows & k) == 0)
            masks.append((bit_clear, jnp.logical_not(bit_clear), pat))
    return masks


def _apply_stage(x, CH, s, mode, mask):
    if s >= 8:
        nb = CH // (2 * s)
        x4 = x.reshape(nb, 2, s, _LANES)
        u = x4[:, 0]
        v = x4[:, 1]
        mn = jnp.minimum(u, v)
        mx = jnp.maximum(u, v)
        if mode == "pat":
            nu = jnp.where(mask, mn, mx)
            nv = jnp.where(mask, mx, mn)
        elif mode == "asc":
            nu, nv = mn, mx
        else:
            nu, nv = mx, mn
        y = jnp.concatenate([nu[:, None], nv[:, None]], axis=1)
        return y.reshape(CH, _LANES)
    bit_clear, not_bit_clear, pat = mask
    p = jnp.where(bit_clear, jnp.roll(x, -s, axis=0), jnp.roll(x, s, axis=0))
    if mode == "pat":
        take_min = pat
    elif mode == "asc":
        take_min = bit_clear
    else:
        take_min = not_bit_clear
    return jnp.where(take_min, jnp.minimum(x, p), jnp.maximum(x, p))


def _far_stage(scr, N, CH, k, s):
    ratio = s // CH
    w = k // (2 * s)

    def run(asc, full):
        def body(t, carry):
            qp = t // ratio
            r = t - qp * ratio
            if full:
                q = qp
            else:
                qh = qp // w
                q = qh * (2 * w) + (qp - qh * w)
                if not asc:
                    q = q + w
            u_off = q * (2 * s) + r * CH
            v_off = u_off + s
            u = scr[pl.ds(u_off, CH), :]
            v = scr[pl.ds(v_off, CH), :]
            mn = jnp.minimum(u, v)
            mx = jnp.maximum(u, v)
            if asc:
                scr[pl.ds(u_off, CH), :] = mn
                scr[pl.ds(v_off, CH), :] = mx
            else:
                scr[pl.ds(u_off, CH), :] = mx
                scr[pl.ds(v_off, CH), :] = mn
            return carry

        trips = N // (2 * CH) if full else N // (4 * CH)
        lax.fori_loop(0, trips, body, 0)

    if k == N:
        run(True, True)
    else:
        run(True, False)
        run(False, False)


def _far_pair(scr, N, CHF, k, s1):
    s2 = s1 // 2
    nq = s1 // (2 * CHF)
    nc = s2 // CHF
    w = k // (2 * s1)

    def run(asc, full, trips):
        def body(t, carry):
            qp = t // nq
            rem = t - qp * nq
            rr = rem // nc
            cc = rem - rr * nc
            if full:
                q = qp
            else:
                qh = qp // w
                q = qh * (2 * w) + (qp - qh * w)
                if not asc:
                    q = q + w
            base = q * (2 * s1) + rr * (2 * s2) + cc * CHF
            a = scr[pl.ds(base, CHF), :]
            b = scr[pl.ds(base + s2, CHF), :]
            c = scr[pl.ds(base + s1, CHF), :]
            d = scr[pl.ds(base + s1 + s2, CHF), :]
            if asc:
                a, c = jnp.minimum(a, c), jnp.maximum(a, c)
                b, d = jnp.minimum(b, d), jnp.maximum(b, d)
                a, b = jnp.minimum(a, b), jnp.maximum(a, b)
                c, d = jnp.minimum(c, d), jnp.maximum(c, d)
            else:
                a, c = jnp.maximum(a, c), jnp.minimum(a, c)
                b, d = jnp.maximum(b, d), jnp.minimum(b, d)
                a, b = jnp.maximum(a, b), jnp.minimum(a, b)
                c, d = jnp.maximum(c, d), jnp.minimum(c, d)
            scr[pl.ds(base, CHF), :] = a
            scr[pl.ds(base + s2, CHF), :] = b
            scr[pl.ds(base + s1, CHF), :] = c
            scr[pl.ds(base + s1 + s2, CHF), :] = d
            return carry

        lax.fori_loop(0, trips, body, 0)

    if k == N:
        run(True, True, N // (4 * CHF))
    else:
        run(True, False, N // (8 * CHF))
        run(False, False, N // (8 * CHF))


def _chunk_pass(scr, N, CH, stages, k_dir, first_mul=None, epilogue=None):
    masks = _stage_masks(CH, stages)
    w = k_dir // CH

    def run(dir_mode, trips, tmap):
        def body(t_p, carry):
            t = tmap(t_p)
            off = t * CH
            if first_mul is None:
                x = scr[pl.ds(off, CH), :]
            else:
                x = first_mul(off, CH)
            for (k, s), mask in zip(stages, masks):
                mode = dir_mode if k >= CH else "pat"
                x = _apply_stage(x, CH, s, mode, mask)
            if epilogue is None:
                scr[pl.ds(off, CH), :] = x
            else:
                epilogue(off, x)
            return carry

        lax.fori_loop(0, trips, body, 0)

    if k_dir == N:
        run("asc", N // CH, lambda t_p: t_p)
    else:
        run("asc", N // (2 * CH),
            lambda t_p: (t_p // w) * (2 * w) + (t_p - (t_p // w) * w))
        run("desc", N // (2 * CH),
            lambda t_p: (t_p // w) * (2 * w) + (t_p - (t_p // w) * w) + w)


def _sort_cols(scr, N, CH, first_mul):
    init = []
    k = 2
    while k <= min(CH, N):
        s = k // 2
        while s > 0:
            init.append((k, s))
            s //= 2
        k *= 2
    _chunk_pass(scr, N, CH, init, min(CH, N), first_mul=first_mul)
    tail = []
    while k <= N:
        far = []
        s = k // 2
        while s >= CH:
            far.append(s)
            s //= 2
        i = 0
        while i + 1 < len(far):
            _far_pair(scr, N, min(128, CH), k, far[i])
            i += 2
        if i < len(far):
            _far_stage(scr, N, CH, k, far[i])
        tail = []
        while s > 0:
            tail.append((k, s))
            s //= 2
        if k == N:
            return tail
        _chunk_pass(scr, N, CH, tail, k)
        k *= 2
    return tail


def _swd_kernel(N, CH, x_ref, m_ref, out_ref, scr_cur, scr_keep):
    j = pl.program_id(0) % 2

    def first_mul(off, ch):
        return x_ref[0, pl.ds(off, ch), :] * m_ref[pl.ds(off, ch), :]

    tail = _sort_cols(scr_cur, N, CH, first_mul)

    @pl.when(j == 0)
    def _():
        def epi(off, x):
            scr_keep[pl.ds(off, CH), :] = x

        _chunk_pass(scr_cur, N, CH, tail, N, epilogue=epi)

    @pl.when(j == 1)
    def _():
        out_ref[0] = jnp.zeros((1, _LANES), jnp.float32)

        def epi(off, x):
            d = jnp.abs(x - scr_keep[pl.ds(off, CH), :])
            out_ref[0] += jnp.sum(d, axis=0, keepdims=True)

        _chunk_pass(scr_cur, N, CH, tail, N, epilogue=epi)


def _scale_colsums(f1, f2, um, CH=256):
    B, C, h, w = f1.shape
    stride = um.shape[1] // h
    N = h * w
    BC = B * C
    m = um[:, ::stride, ::stride].reshape(B, N)
    m_bc = jnp.repeat(m.T, C, axis=1)
    a_t = f1.reshape(BC, N).T
    b_t = f2.reshape(BC, N).T
    x = jnp.stack([a_t, b_t])
    ncb = BC // _LANES
    out = pl.pallas_call(
        functools.partial(_swd_kernel, N, CH),
        grid=(2 * ncb,),
        in_specs=[
            pl.BlockSpec((1, N, _LANES), lambda g: (g % 2, 0, g // 2)),
            pl.BlockSpec((N, _LANES), lambda g: (0, g // 2)),
        ],
        out_specs=pl.BlockSpec((1, 1, _LANES), lambda g: (g // 2, 0, 0)),
        out_shape=jax.ShapeDtypeStruct((ncb, 1, _LANES), jnp.float32),
        scratch_shapes=[
            pltpu.VMEM((N, _LANES), jnp.float32),
            pltpu.VMEM((N, _LANES), jnp.float32),
        ],
    )(x, m_bc)
    return out.reshape(BC), m, N


@jax.jit
def kernel(feat_t1_s0, feat_t1_s1, feat_t2_s0, feat_t2_s1, target_mask):
    um = (1 - target_mask).astype(jnp.float32)
    losses = []
    for f1, f2 in ((feat_t1_s0, feat_t2_s0), (feat_t1_s1, feat_t2_s1)):
        colsums, m, N = _scale_colsums(f1, f2, um)
        B, C = f1.shape[0], f1.shape[1]
        valid = jnp.maximum(jnp.sum(m, axis=1), 1.0)
        per_b = colsums.reshape(B, C).sum(axis=1) / valid
        losses.append(jnp.sum(per_b) / (B * C * N))
    return (losses[0] + losses[1]) * 0.5


# speedup vs baseline: 1.2907x; 1.0148x over previous
import functools

import numpy as np
import jax
import jax.numpy as jnp
from jax import lax
from jax.experimental import pallas as pl
from jax.experimental.pallas import tpu as pltpu

_LANES = 128


def _stage_masks(CH, stages):
    masks = []
    for (k, s) in stages:
        if s >= 8:
            if k >= CH:
                masks.append(None)
            else:
                nb = CH // (2 * s)
                blk = lax.broadcasted_iota(jnp.int32, (nb, 1, _LANES), 0)
                masks.append(((blk * (2 * s)) & k) == 0)
        else:
            rows = lax.broadcasted_iota(jnp.int32, (CH, _LANES), 0)
            bit_clear = (rows & s) == 0
            if k >= CH:
                pat = None
            else:
                pat = bit_clear == ((rows & k) == 0)
            masks.append((bit_clear, jnp.logical_not(bit_clear), pat))
    return masks


def _apply_stage(x, CH, s, mode, mask):
    if s >= 8:
        nb = CH // (2 * s)
        x4 = x.reshape(nb, 2, s, _LANES)
        u = x4[:, 0]
        v = x4[:, 1]
        mn = jnp.minimum(u, v)
        mx = jnp.maximum(u, v)
        if mode == "pat":
            nu = jnp.where(mask, mn, mx)
            nv = jnp.where(mask, mx, mn)
        elif mode == "asc":
            nu, nv = mn, mx
        else:
            nu, nv = mx, mn
        y = jnp.concatenate([nu[:, None], nv[:, None]], axis=1)
        return y.reshape(CH, _LANES)
    bit_clear, not_bit_clear, pat = mask
    p = jnp.where(bit_clear, jnp.roll(x, -s, axis=0), jnp.roll(x, s, axis=0))
    if mode == "pat":
        take_min = pat
    elif mode == "asc":
        take_min = bit_clear
    else:
        take_min = not_bit_clear
    return jnp.where(take_min, jnp.minimum(x, p), jnp.maximum(x, p))


def _far_stage(scr, N, CH, k, s):
    ratio = s // CH
    w = k // (2 * s)

    def run(asc, full):
        def body(t, carry):
            qp = t // ratio
            r = t - qp * ratio
            if full:
                q = qp
            else:
                qh = qp // w
                q = qh * (2 * w) + (qp - qh * w)
                if not asc:
                    q = q + w
            u_off = q * (2 * s) + r * CH
            v_off = u_off + s
            u = scr[pl.ds(u_off, CH), :]
            v = scr[pl.ds(v_off, CH), :]
            mn = jnp.minimum(u, v)
            mx = jnp.maximum(u, v)
            if asc:
                scr[pl.ds(u_off, CH), :] = mn
                scr[pl.ds(v_off, CH), :] = mx
            else:
                scr[pl.ds(u_off, CH), :] = mx
                scr[pl.ds(v_off, CH), :] = mn
            return carry

        trips = N // (2 * CH) if full else N // (4 * CH)
        lax.fori_loop(0, trips, body, 0)

    if k == N:
        run(True, True)
    else:
        run(True, False)
        run(False, False)


def _far_group(scr, N, CHF, k, s_top, g):
    strides = [s_top >> i for i in range(g)]
    sg = strides[-1]
    nc = sg // CHF
    w = k // (2 * s_top)
    nslab = 1 << g

    def run(asc, full, trips):
        def body(t, carry):
            qp = t // nc
            cc = t - qp * nc
            if full:
                q = qp
            else:
                qh = qp // w
                q = qh * (2 * w) + (qp - qh * w)
                if not asc:
                    q = q + w
            base = q * (2 * s_top) + cc * CHF
            offs = []
            for j in range(nslab):
                o = base
                for i in range(g):
                    if (j >> i) & 1:
                        o = o + strides[i]
                offs.append(o)
            slabs = [scr[pl.ds(o, CHF), :] for o in offs]
            for i in range(g):
                bit = 1 << i
                for j in range(nslab):
                    if not (j & bit):
                        u, v = slabs[j], slabs[j | bit]
                        if asc:
                            slabs[j] = jnp.minimum(u, v)
                            slabs[j | bit] = jnp.maximum(u, v)
                        else:
                            slabs[j] = jnp.maximum(u, v)
                            slabs[j | bit] = jnp.minimum(u, v)
            for j in range(nslab):
                scr[pl.ds(offs[j], CHF), :] = slabs[j]
            return carry

        lax.fori_loop(0, trips, body, 0)

    if k == N:
        run(True, True, N // (nslab * CHF))
    else:
        run(True, False, N // (2 * nslab * CHF))
        run(False, False, N // (2 * nslab * CHF))


def _chunk_pass(scr, N, CH, stages, k_dir, first_mul=None, epilogue=None):
    masks = _stage_masks(CH, stages)
    w = k_dir // CH

    def run(dir_mode, trips, tmap):
        def body(t_p, carry):
            t = tmap(t_p)
            off = t * CH
            if first_mul is None:
                x = scr[pl.ds(off, CH), :]
            else:
                x = first_mul(off, CH)
            for (k, s), mask in zip(stages, masks):
                mode = dir_mode if k >= CH else "pat"
                x = _apply_stage(x, CH, s, mode, mask)
            if epilogue is None:
                scr[pl.ds(off, CH), :] = x
            else:
                epilogue(off, x)
            return carry

        lax.fori_loop(0, trips, body, 0)

    if k_dir == N:
        run("asc", N // CH, lambda t_p: t_p)
    else:
        run("asc", N // (2 * CH),
            lambda t_p: (t_p // w) * (2 * w) + (t_p - (t_p // w) * w))
        run("desc", N // (2 * CH),
            lambda t_p: (t_p // w) * (2 * w) + (t_p - (t_p // w) * w) + w)


def _sort_cols(scr, N, CH, first_mul):
    init = []
    k = 2
    while k <= min(CH, N):
        s = k // 2
        while s > 0:
            init.append((k, s))
            s //= 2
        k *= 2
    _chunk_pass(scr, N, CH, init, min(CH, N), first_mul=first_mul)
    tail = []
    while k <= N:
        far = []
        s = k // 2
        while s >= CH:
            far.append(s)
            s //= 2
        i = 0
        while len(far) - i >= 3:
            _far_group(scr, N, min(64, CH), k, far[i], 3)
            i += 3
        if len(far) - i == 2:
            _far_group(scr, N, min(128, CH), k, far[i], 2)
            i += 2
        elif len(far) - i == 1:
            _far_stage(scr, N, CH, k, far[i])
        tail = []
        while s > 0:
            tail.append((k, s))
            s //= 2
        if k == N:
            return tail
        _chunk_pass(scr, N, CH, tail, k)
        k *= 2
    return tail


def _swd_kernel(N, CH, x_ref, m_ref, out_ref, scr_cur, scr_keep):
    j = pl.program_id(0) % 2

    def first_mul(off, ch):
        return x_ref[0, pl.ds(off, ch), :] * m_ref[pl.ds(off, ch), :]

    tail = _sort_cols(scr_cur, N, CH, first_mul)

    @pl.when(j == 0)
    def _():
        def epi(off, x):
            scr_keep[pl.ds(off, CH), :] = x

        _chunk_pass(scr_cur, N, CH, tail, N, epilogue=epi)

    @pl.when(j == 1)
    def _():
        out_ref[0] = jnp.zeros((1, _LANES), jnp.float32)

        def epi(off, x):
            d = jnp.abs(x - scr_keep[pl.ds(off, CH), :])
            out_ref[0] += jnp.sum(d, axis=0, keepdims=True)

        _chunk_pass(scr_cur, N, CH, tail, N, epilogue=epi)


def _scale_colsums(f1, f2, um, CH=256):
    B, C, h, w = f1.shape
    stride = um.shape[1] // h
    N = h * w
    BC = B * C
    m = um[:, ::stride, ::stride].reshape(B, N)
    m_bc = jnp.repeat(m.T, C, axis=1)
    a_t = f1.reshape(BC, N).T
    b_t = f2.reshape(BC, N).T
    x = jnp.stack([a_t, b_t])
    ncb = BC // _LANES
    out = pl.pallas_call(
        functools.partial(_swd_kernel, N, CH),
        grid=(2 * ncb,),
        in_specs=[
            pl.BlockSpec((1, N, _LANES), lambda g: (g % 2, 0, g // 2)),
            pl.BlockSpec((N, _LANES), lambda g: (0, g // 2)),
        ],
        out_specs=pl.BlockSpec((1, 1, _LANES), lambda g: (g // 2, 0, 0)),
        out_shape=jax.ShapeDtypeStruct((ncb, 1, _LANES), jnp.float32),
        scratch_shapes=[
            pltpu.VMEM((N, _LANES), jnp.float32),
            pltpu.VMEM((N, _LANES), jnp.float32),
        ],
    )(x, m_bc)
    return out.reshape(BC), m, N


@jax.jit
def kernel(feat_t1_s0, feat_t1_s1, feat_t2_s0, feat_t2_s1, target_mask):
    um = (1 - target_mask).astype(jnp.float32)
    losses = []
    for f1, f2 in ((feat_t1_s0, feat_t2_s0), (feat_t1_s1, feat_t2_s1)):
        colsums, m, N = _scale_colsums(f1, f2, um)
        B, C = f1.shape[0], f1.shape[1]
        valid = jnp.maximum(jnp.sum(m, axis=1), 1.0)
        per_b = colsums.reshape(B, C).sum(axis=1) / valid
        losses.append(jnp.sum(per_b) / (B * C * N))
    return (losses[0] + losses[1]) * 0.5
